# baseline (speedup 1.0000x reference)
import jax
import jax.numpy as jnp
from jax import lax
from jax.experimental import pallas as pl
from jax.experimental.pallas import tpu as pltpu
from jax.experimental.pallas import tpu_sc as plsc

F32 = jnp.float32
I32 = jnp.int32

N = 10000
NPAD = 10240
NSLICE = NPAD // 16
E = 160000
ET = E + N
ETP = 172032
EW = ETP // 32
NCH = EW // 16
ELP = 161792
EWL = ELP // 32
NCHL = EWL // 16

DW = 128
NJ = DW // 16


def _mesh():
    return plsc.VectorSubcoreMesh(core_axis_name="c", subcore_axis_name="s")


def _sc_params():
    return pltpu.CompilerParams(needs_layout_passes=False)


def _worker_id():
    return lax.axis_index("c") * 16 + lax.axis_index("s")


def _zero_1d(ref, nwords):
    zero = jnp.zeros((16,), F32)

    def zb(k, carry):
        ref[pl.ds(k * 16, 16)] = zero
        return carry

    lax.fori_loop(0, nwords // 16, zb, None)


def _combine_to_hbm(local_ref, shs, tmp_v, red_v, out_hbm):
    c = lax.axis_index("c")
    sb = lax.axis_index("s")
    pltpu.sync_copy(local_ref, shs.at[sb])
    plsc.subcore_barrier()
    off = sb * NSLICE
    pltpu.sync_copy(shs.at[0, pl.ds(off, NSLICE)], red_v)
    for r in range(1, 16):
        pltpu.sync_copy(shs.at[r, pl.ds(off, NSLICE)], tmp_v)

        def addk(k, carry):
            red_v[pl.ds(k * 16, 16)] = (
                red_v[pl.ds(k * 16, 16)] + tmp_v[pl.ds(k * 16, 16)]
            )
            return carry

        lax.fori_loop(0, NSLICE // 16, addk, None)
    pltpu.sync_copy(red_v, out_hbm.at[c, pl.ds(off, NSLICE)])


def _load_s_tot(sp_hbm, s_tot, tmp_big):
    pltpu.sync_copy(sp_hbm.at[0], s_tot)
    pltpu.sync_copy(sp_hbm.at[1], tmp_big)

    def sk(k, carry):
        s_tot[pl.ds(k * 16, 16)] = (
            s_tot[pl.ds(k * 16, 16)] + tmp_big[pl.ds(k * 16, 16)] + 1e-16
        )
        return carry

    lax.fori_loop(0, NPAD // 16, sk, None)


NBUF = 4


def _gat_stage1(src_p, dst_p, att_p, pairs):
    npairs = len(pairs)
    nt = 2 * npairs
    NH = npairs * DW
    nacts = [p[2] for p in pairs]

    def body(src_hbm, dst_hbm, att_hbm, *rest):
        tabs = rest[:nt]
        ex_hbm, spart_hbm = rest[nt], rest[nt + 1]
        scr = rest[nt + 2:]
        att_v, src_big, dst_big, ex_big = scr[0], scr[1], scr[2], scr[3]
        rows = [scr[4 + b * nt:4 + (b + 1) * nt] for b in range(NBUF)]
        rest2 = scr[4 + NBUF * nt:]
        P, s_loc, tmp_v, red_v, shs = rest2[:5]
        sems = rest2[5:5 + NBUF]

        wid = _worker_id()
        wbase = wid * EW
        pltpu.sync_copy(att_hbm, att_v)
        pltpu.sync_copy(src_hbm.at[pl.ds(wbase, EW)], src_big)
        pltpu.sync_copy(dst_hbm.at[pl.ds(wbase, EW)], dst_big)
        _zero_1d(s_loc, NPAD)
        att_regs = [att_v[pl.ds(k * 16, 16)] for k in range(NH // 16)]
        iot = lax.iota(I32, 16)

        def issue(k, b):
            sreg = src_big[pl.ds(k * 16, 16)]
            dreg = dst_big[pl.ds(k * 16, 16)]
            for p in range(npairs):
                pltpu.async_copy(tabs[2 * p].at[sreg], rows[b][2 * p], sems[b])
                pltpu.async_copy(tabs[2 * p + 1].at[dreg], rows[b][2 * p + 1],
                                 sems[b])

        def wait(b):
            for p in range(nt):
                pltpu.make_async_copy(tabs[p].at[iot], rows[b][p],
                                      sems[b]).wait()

        for b in range(NBUF):
            issue(b, b)

        def group(g, carry):
            for b in range(NBUF):
                k = g * NBUF + b
                wait(b)

                def edge(e, ecarry):
                    acc0 = jnp.zeros((16,), F32)
                    acc1 = jnp.zeros((16,), F32)
                    for p in range(npairs):
                        for j in range(nacts[p]):
                            v = (rows[b][2 * p][e, pl.ds(j * 16, 16)]
                                 + rows[b][2 * p + 1][e, pl.ds(j * 16, 16)])
                            v = jnp.where(v >= 0, v, 0.2 * v)
                            if j % 2 == 0:
                                acc0 = acc0 + v * att_regs[p * NJ + j]
                            else:
                                acc1 = acc1 + v * att_regs[p * NJ + j]
                    P[pl.ds(e * 16, 16)] = acc0 + acc1
                    return ecarry

                lax.fori_loop(0, 16, edge, None)
                t0 = jnp.zeros((16,), F32)
                t1 = jnp.zeros((16,), F32)
                for col in range(0, 16, 2):
                    t0 = t0 + plsc.load_gather(P, [iot * 16 + col])
                    t1 = t1 + plsc.load_gather(P, [iot * 16 + (col + 1)])
                base = wbase + k * 16
                mask = (base + iot) < ET
                exv = jnp.where(mask, jnp.exp(t0 + t1), 0.0)
                ex_big[pl.ds(k * 16, 16)] = exv
                plsc.addupdate_scatter(s_loc, [dst_big[pl.ds(k * 16, 16)]], exv)

                @pl.when(k + NBUF < NCH)
                def _():
                    issue(k + NBUF, b)
            return carry

        lax.fori_loop(0, NCH // NBUF, group, None)
        pltpu.sync_copy(ex_big, ex_hbm.at[pl.ds(wbase, EW)])
        _combine_to_hbm(s_loc, shs, tmp_v, red_v, spart_hbm)

    scratch = (
        [pltpu.VMEM((NH,), F32), pltpu.VMEM((EW,), I32), pltpu.VMEM((EW,), I32),
         pltpu.VMEM((EW,), F32)]
        + [pltpu.VMEM((16, DW), F32)] * (NBUF * nt)
        + [pltpu.VMEM((256,), F32),
           pltpu.VMEM((NPAD,), F32), pltpu.VMEM((NSLICE,), F32),
           pltpu.VMEM((NSLICE,), F32),
           pltpu.VMEM_SHARED((16, NPAD), F32)]
        + [pltpu.SemaphoreType.DMA] * NBUF
    )
    flat_tabs = [a for pair in pairs for a in pair[:2]]
    fn = pl.kernel(
        body,
        out_type=[jax.ShapeDtypeStruct((ETP,), F32),
                  jax.ShapeDtypeStruct((2, NPAD), F32)],
        mesh=_mesh(),
        compiler_params=_sc_params(),
        scratch_types=scratch,
    )
    return fn(src_p, dst_p, att_p, *flat_tabs)


def _gat_stage2(src_p, dst_p, ex, spart, xl, zeros_hbm, nact):
    def body(src_hbm, dst_hbm, ex_hbm, sp_hbm, xl_hbm, z_hbm, op_hbm,
             s_tot, tmp_big, src_big, dst_big, ex_big,
             rows0, rows1, scat0, scat1, a_buf, osh,
             sem0, sem1, ssem0, ssem1):
        c = lax.axis_index("c")
        sb = lax.axis_index("s")
        wid = _worker_id()
        wbase = wid * EW
        rows = (rows0, rows1)
        scat = (scat0, scat1)
        sems = (sem0, sem1)
        ssems = (ssem0, ssem1)
        _load_s_tot(sp_hbm, s_tot, tmp_big)
        pltpu.sync_copy(src_hbm.at[pl.ds(wbase, EW)], src_big)
        pltpu.sync_copy(dst_hbm.at[pl.ds(wbase, EW)], dst_big)
        pltpu.sync_copy(ex_hbm.at[pl.ds(wbase, EW)], ex_big)
        iot = lax.iota(I32, 16)

        pltpu.sync_copy(z_hbm, osh.at[pl.ds(sb * NSLICE, NSLICE)])
        plsc.subcore_barrier()

        def issue(k, b):
            pltpu.async_copy(xl_hbm.at[src_big[pl.ds(k * 16, 16)]], rows[b],
                             sems[b])

        issue(0, 0)
        issue(1, 1)

        def group(g, carry):
            for b in (0, 1):
                k = g * 2 + b
                pltpu.make_async_copy(xl_hbm.at[iot], rows[b], sems[b]).wait()
                dreg = dst_big[pl.ds(k * 16, 16)]
                sv = plsc.load_gather(s_tot, [dreg])
                a_buf[...] = ex_big[pl.ds(k * 16, 16)] / sv

                @pl.when(k >= 2)
                def _():
                    pltpu.make_async_copy(scat[b], osh.at[iot],
                                          ssems[b]).wait()

                def edge(e, ecarry):
                    av = plsc.load_gather(a_buf, [jnp.full((16,), 0, I32) + e])
                    for j in range(nact):
                        scat[b][e, pl.ds(j * 16, 16)] = (
                            rows[b][e, pl.ds(j * 16, 16)] * av)
                    return ecarry

                lax.fori_loop(0, 16, edge, None)
                pltpu.async_copy(scat[b], osh.at[dreg], ssems[b], add=True)

                @pl.when(k + 2 < NCH)
                def _():
                    issue(k + 2, b)
            return carry

        lax.fori_loop(0, NCH // 2, group, None)
        pltpu.make_async_copy(scat[0], osh.at[iot], ssems[0]).wait()
        pltpu.make_async_copy(scat[1], osh.at[iot], ssems[1]).wait()
        plsc.subcore_barrier()
        pltpu.sync_copy(osh.at[pl.ds(sb * NSLICE, NSLICE)],
                        op_hbm.at[c, pl.ds(sb * NSLICE, NSLICE)])

    def zero_fill(ref):
        zero = jnp.zeros((16,), F32)
        for r in range(16):
            for j in range(NJ):
                ref[r, pl.ds(j * 16, 16)] = zero

    real_body = body

    def body_with_init(*args):
        scat0, scat1 = args[14], args[15]
        zero_fill(scat0)
        zero_fill(scat1)
        real_body(*args)

    scratch = [
        pltpu.VMEM((NPAD,), F32), pltpu.VMEM((NPAD,), F32),
        pltpu.VMEM((EW,), I32), pltpu.VMEM((EW,), I32), pltpu.VMEM((EW,), F32),
        pltpu.VMEM((16, DW), F32), pltpu.VMEM((16, DW), F32),
        pltpu.VMEM((16, DW), F32), pltpu.VMEM((16, DW), F32),
        pltpu.VMEM((16,), F32),
        pltpu.VMEM_SHARED((NPAD, DW), F32),
        pltpu.SemaphoreType.DMA, pltpu.SemaphoreType.DMA,
        pltpu.SemaphoreType.DMA, pltpu.SemaphoreType.DMA,
    ]
    fn = pl.kernel(
        body_with_init,
        out_type=jax.ShapeDtypeStruct((2, NPAD, DW), F32),
        mesh=_mesh(),
        compiler_params=_sc_params(),
        scratch_types=scratch,
    )
    return fn(src_p, dst_p, ex, spart, xl, zeros_hbm)


def _l3_stage1(src_p, dst_p, xl3, xr3, att3b):
    def body(src_hbm, dst_hbm, xl_hbm, xr_hbm, att_hbm, ex_hbm, spart_hbm,
             xl_v, xr_v, att_v, src_big, dst_big, ex_big, s_loc, tmp_v, red_v,
             shs, sem):
        wid = _worker_id()
        wbase = wid * EW
        pltpu.sync_copy(xl_hbm, xl_v)
        pltpu.sync_copy(xr_hbm, xr_v)
        pltpu.sync_copy(att_hbm, att_v)
        pltpu.sync_copy(src_hbm.at[pl.ds(wbase, EW)], src_big)
        pltpu.sync_copy(dst_hbm.at[pl.ds(wbase, EW)], dst_big)
        _zero_1d(s_loc, NPAD)
        att_reg = att_v[...]
        iot = lax.iota(I32, 16)

        def chunk(i, carry):
            sreg = src_big[pl.ds(i * 16, 16)]
            dreg = dst_big[pl.ds(i * 16, 16)]
            v = plsc.load_gather(xl_v, [sreg]) + plsc.load_gather(xr_v, [dreg])
            t = jnp.where(v >= 0, v, 0.2 * v) * att_reg
            mask = (wbase + i * 16 + iot) < ET
            exv = jnp.where(mask, jnp.exp(t), 0.0)
            ex_big[pl.ds(i * 16, 16)] = exv
            plsc.addupdate_scatter(s_loc, [dreg], exv)
            return carry

        lax.fori_loop(0, NCH, chunk, None)
        pltpu.sync_copy(ex_big, ex_hbm.at[pl.ds(wbase, EW)])
        _combine_to_hbm(s_loc, shs, tmp_v, red_v, spart_hbm)

    scratch = [
        pltpu.VMEM((NPAD,), F32), pltpu.VMEM((NPAD,), F32),
        pltpu.VMEM((16,), F32),
        pltpu.VMEM((EW,), I32), pltpu.VMEM((EW,), I32), pltpu.VMEM((EW,), F32),
        pltpu.VMEM((NPAD,), F32), pltpu.VMEM((NSLICE,), F32),
        pltpu.VMEM((NSLICE,), F32),
        pltpu.VMEM_SHARED((16, NPAD), F32), pltpu.SemaphoreType.DMA,
    ]
    fn = pl.kernel(
        body,
        out_type=[jax.ShapeDtypeStruct((ETP,), F32),
                  jax.ShapeDtypeStruct((2, NPAD), F32)],
        mesh=_mesh(),
        compiler_params=_sc_params(),
        scratch_types=scratch,
    )
    return fn(src_p, dst_p, xl3, xr3, att3b)


def _l3_stage2(src_p, dst_p, ex3, spart, xl3):
    def body(src_hbm, dst_hbm, ex_hbm, sp_hbm, xl_hbm, op_hbm,
             xl_v, s_tot, tmp_big, src_big, dst_big, ex_big, o_loc, tmp_v,
             red_v, shs, sem):
        wid = _worker_id()
        wbase = wid * EW
        pltpu.sync_copy(xl_hbm, xl_v)
        _load_s_tot(sp_hbm, s_tot, tmp_big)
        pltpu.sync_copy(src_hbm.at[pl.ds(wbase, EW)], src_big)
        pltpu.sync_copy(dst_hbm.at[pl.ds(wbase, EW)], dst_big)
        pltpu.sync_copy(ex_hbm.at[pl.ds(wbase, EW)], ex_big)
        _zero_1d(o_loc, NPAD)

        def chunk(i, carry):
            sreg = src_big[pl.ds(i * 16, 16)]
            dreg = dst_big[pl.ds(i * 16, 16)]
            alpha = ex_big[pl.ds(i * 16, 16)] / plsc.load_gather(s_tot, [dreg])
            o = alpha * plsc.load_gather(xl_v, [sreg])
            plsc.addupdate_scatter(o_loc, [dreg], o)
            return carry

        lax.fori_loop(0, NCH, chunk, None)
        _combine_to_hbm(o_loc, shs, tmp_v, red_v, op_hbm)

    scratch = [
        pltpu.VMEM((NPAD,), F32), pltpu.VMEM((NPAD,), F32),
        pltpu.VMEM((NPAD,), F32),
        pltpu.VMEM((EW,), I32), pltpu.VMEM((EW,), I32), pltpu.VMEM((EW,), F32),
        pltpu.VMEM((NPAD,), F32), pltpu.VMEM((NSLICE,), F32),
        pltpu.VMEM((NSLICE,), F32),
        pltpu.VMEM_SHARED((16, NPAD), F32), pltpu.SemaphoreType.DMA,
    ]
    fn = pl.kernel(
        body,
        out_type=jax.ShapeDtypeStruct((2, NPAD), F32),
        mesh=_mesh(),
        compiler_params=_sc_params(),
        scratch_types=scratch,
    )
    return fn(src_p, dst_p, ex3, spart, xl3)


def _loss_dots(pa, pb, na, nb, z, nact=7):
    def body(pa_hbm, pb_hbm, na_hbm, nb_hbm, z_hbm, dp_hbm, dn_hbm,
             a_big, b_big, d_big, *rest):
        za = rest[:NBUF]
        zb = rest[NBUF:2 * NBUF]
        P = rest[2 * NBUF]
        sems = rest[2 * NBUF + 1:]
        wid = _worker_id()
        wbase = wid * EWL
        iot = lax.iota(I32, 16)
        for (a_hbm, b_hbm, o_hbm) in ((pa_hbm, pb_hbm, dp_hbm),
                                      (na_hbm, nb_hbm, dn_hbm)):
            pltpu.sync_copy(a_hbm.at[pl.ds(wbase, EWL)], a_big)
            pltpu.sync_copy(b_hbm.at[pl.ds(wbase, EWL)], b_big)

            def issue(k, b):
                pltpu.async_copy(z_hbm.at[a_big[pl.ds(k * 16, 16)]], za[b],
                                 sems[b])
                pltpu.async_copy(z_hbm.at[b_big[pl.ds(k * 16, 16)]], zb[b],
                                 sems[b])

            for b in range(NBUF):
                issue(b, b)

            def group(g, carry):
                for b in range(NBUF):
                    k = g * NBUF + b
                    pltpu.make_async_copy(z_hbm.at[iot], za[b], sems[b]).wait()
                    pltpu.make_async_copy(z_hbm.at[iot], zb[b], sems[b]).wait()

                    def edge(e, ecarry):
                        acc0 = jnp.zeros((16,), F32)
                        acc1 = jnp.zeros((16,), F32)
                        for j in range(nact):
                            t = (za[b][e, pl.ds(j * 16, 16)]
                                 * zb[b][e, pl.ds(j * 16, 16)])
                            if j % 2 == 0:
                                acc0 = acc0 + t
                            else:
                                acc1 = acc1 + t
                        P[pl.ds(e * 16, 16)] = acc0 + acc1
                        return ecarry

                    lax.fori_loop(0, 16, edge, None)
                    t0 = jnp.zeros((16,), F32)
                    t1 = jnp.zeros((16,), F32)
                    for col in range(0, 16, 2):
                        t0 = t0 + plsc.load_gather(P, [iot * 16 + col])
                        t1 = t1 + plsc.load_gather(P, [iot * 16 + (col + 1)])
                    d_big[pl.ds(k * 16, 16)] = t0 + t1

                    @pl.when(k + NBUF < NCHL)
                    def _():
                        issue(k + NBUF, b)
                return carry

            lax.fori_loop(0, NCHL // NBUF, group, None)
            pltpu.sync_copy(d_big, o_hbm.at[pl.ds(wbase, EWL)])

    scratch = (
        [pltpu.VMEM((EWL,), I32), pltpu.VMEM((EWL,), I32),
         pltpu.VMEM((EWL,), F32)]
        + [pltpu.VMEM((16, DW), F32)] * (2 * NBUF)
        + [pltpu.VMEM((256,), F32)]
        + [pltpu.SemaphoreType.DMA] * NBUF
    )
    fn = pl.kernel(
        body,
        out_type=[jax.ShapeDtypeStruct((ELP,), F32),
                  jax.ShapeDtypeStruct((ELP,), F32)],
        mesh=_mesh(),
        compiler_params=_sc_params(),
        scratch_types=scratch,
    )
    return fn(pa, pb, na, nb, z)


_RB = 2000


def _t1(x, ws):
    nw = len(ws)

    def body(*refs):
        x_ref = refs[0]
        w_refs = refs[1:1 + nw]
        o_refs = refs[1 + nw:]
        xb = x_ref[...]
        for w, o in zip(w_refs, o_refs):
            o[...] = jnp.dot(xb, w[...], preferred_element_type=F32)

    return pl.pallas_call(
        body,
        grid=(N // _RB,),
        in_specs=[pl.BlockSpec((_RB, 128), lambda i: (i, 0))]
        + [pl.BlockSpec((128, DW), lambda i: (0, 0))] * nw,
        out_specs=[pl.BlockSpec((_RB, DW), lambda i: (i, 0))] * nw,
        out_shape=[jax.ShapeDtypeStruct((N, DW), F32)] * nw,
    )(x, *ws)


def _t2(o1parts, b1p, wl2p, wr2p):
    def body(oa_ref, ob_ref, oc_ref, b1_ref, wl_ref, wr_ref, xl2_ref, xr2_ref):
        h = jnp.concatenate(
            [oa_ref[0] + oa_ref[1], ob_ref[0] + ob_ref[1],
             oc_ref[0] + oc_ref[1]], axis=1)
        h = jnp.maximum(h + b1_ref[...], 0.0)
        xl2_ref[...] = jnp.dot(h, wl_ref[...], preferred_element_type=F32)
        xr2_ref[...] = jnp.dot(h, wr_ref[...], preferred_element_type=F32)

    return pl.pallas_call(
        body,
        grid=(N // _RB,),
        in_specs=[pl.BlockSpec((2, _RB, DW), lambda i: (0, i, 0))] * 3
        + [
            pl.BlockSpec((1, 3 * DW), lambda i: (0, 0)),
            pl.BlockSpec((3 * DW, DW), lambda i: (0, 0)),
            pl.BlockSpec((3 * DW, DW), lambda i: (0, 0)),
        ],
        out_specs=[pl.BlockSpec((_RB, DW), lambda i: (i, 0))] * 2,
        out_shape=[jax.ShapeDtypeStruct((N, DW), F32)] * 2,
    )(*o1parts, b1p, wl2p, wr2p)


def _t3(o2, b2p, x, wlin1p, blin1p, wlin2p, blin2p, w3p):
    def body(o2_ref, b2_ref, x_ref, w1_ref, bl1_ref, w2_ref, bl2_ref, w3_ref,
             z_ref, o3c_ref):
        x1 = jnp.maximum(o2_ref[0] + o2_ref[1] + b2_ref[...], 0.0)
        xb = x_ref[...]
        t1 = jnp.maximum(
            jnp.dot(xb, w1_ref[...], preferred_element_type=F32)
            + bl1_ref[...], 0.0)
        xs = x1 + t1
        t2 = jnp.maximum(
            jnp.dot(xb, w2_ref[...], preferred_element_type=F32)
            + bl2_ref[...], 0.0)
        z_ref[...] = x1 + t2
        o3c_ref[...] = jnp.dot(xs, w3_ref[...], preferred_element_type=F32)

    return pl.pallas_call(
        body,
        grid=(N // _RB,),
        in_specs=[
            pl.BlockSpec((2, _RB, DW), lambda i: (0, i, 0)),
            pl.BlockSpec((1, DW), lambda i: (0, 0)),
            pl.BlockSpec((_RB, 128), lambda i: (i, 0)),
            pl.BlockSpec((128, DW), lambda i: (0, 0)),
            pl.BlockSpec((1, DW), lambda i: (0, 0)),
            pl.BlockSpec((128, DW), lambda i: (0, 0)),
            pl.BlockSpec((1, DW), lambda i: (0, 0)),
            pl.BlockSpec((DW, 128), lambda i: (0, 0)),
        ],
        out_specs=[
            pl.BlockSpec((_RB, DW), lambda i: (i, 0)),
            pl.BlockSpec((_RB, 128), lambda i: (i, 0)),
        ],
        out_shape=[
            jax.ShapeDtypeStruct((N, DW), F32),
            jax.ShapeDtypeStruct((N, 128), F32),
        ],
    )(o2, b2p, x, wlin1p, blin1p, wlin2p, blin2p, w3p)


def _t4(dp2, dn2, p3, b3r):
    def body(dp_ref, dn_ref, p3_ref, b3_ref, rl_ref, o3_ref):
        p = jax.nn.sigmoid(dp_ref[...])
        pls = -jnp.mean(jnp.log(p + 1e-15))
        q = jax.nn.sigmoid(dn_ref[...])
        nls = -jnp.mean(jnp.log(1.0 - q + 1e-15))
        rl_ref[...] = jnp.reshape(pls + nls, (1, 1))
        o3_ref[...] = p3_ref[0] + p3_ref[1] + b3_ref[...]

    return pl.pallas_call(
        body,
        out_shape=[jax.ShapeDtypeStruct((1, 1), F32),
                   jax.ShapeDtypeStruct((NPAD,), F32)],
    )(dp2, dn2, p3, b3r)


def kernel(x, edge_index, neg_edge_index, Wl1, Wr1, att1, b1, Wl2, Wr2, att2,
           b2, Wl3, Wr3, att3, b3, Wlin1, blin1, Wlin2, blin2, c1, c2):
    loop = jnp.arange(N, dtype=edge_index.dtype)
    src = jnp.concatenate([edge_index[0], loop])
    dst = jnp.concatenate([edge_index[1], loop])
    src_p = jnp.pad(src, (0, ETP - ET))
    dst_p = jnp.pad(dst, (0, ETP - ET))

    wl1p = jnp.pad(Wl1, ((0, 0), (0, 84)))
    wr1p = jnp.pad(Wr1, ((0, 0), (0, 84)))
    att1p = jnp.pad(att1, (0, 84))
    b1p = jnp.pad(b1, (0, 84)).reshape(1, 3 * DW)

    t1outs = _t1(x, [wl1p[:, :DW], wl1p[:, DW:2 * DW], wl1p[:, 2 * DW:],
                     wr1p[:, :DW], wr1p[:, DW:2 * DW], wr1p[:, 2 * DW:]])
    xl1s, xr1s = t1outs[:3], t1outs[3:]
    nacts1 = [8, 8, 3]
    pairs1 = [(l, r, na) for (l, r, na) in zip(xl1s, xr1s, nacts1)]
    zeros_hbm = jnp.zeros((NSLICE, DW), F32)
    ex1, sp1 = _gat_stage1(src_p, dst_p, att1p, pairs1)
    o1parts = [_gat_stage2(src_p, dst_p, ex1, sp1, t, zeros_hbm, na)
               for t, na in zip(xl1s, nacts1)]

    wl2p = jnp.pad(Wl2, ((0, 84), (0, 28)))
    wr2p = jnp.pad(Wr2, ((0, 84), (0, 28)))
    att2p = jnp.pad(att2, (0, 28))
    b2p = jnp.pad(b2, (0, 28)).reshape(1, DW)
    xl2, xr2 = _t2(o1parts, b1p, wl2p, wr2p)
    ex2, sp2 = _gat_stage1(src_p, dst_p, att2p, [(xl2, xr2, 7)])
    o2 = _gat_stage2(src_p, dst_p, ex2, sp2, xl2, zeros_hbm, 7)

    wlin1p = jnp.pad(Wlin1, ((0, 0), (0, 28)))
    blin1p = jnp.pad(blin1, (0, 28)).reshape(1, DW)
    wlin2p = jnp.pad(Wlin2, ((0, 0), (0, 28)))
    blin2p = jnp.pad(blin2, (0, 28)).reshape(1, DW)
    w3p = jnp.pad(jnp.concatenate([Wl3, Wr3], axis=1), ((0, 28), (0, 126)))
    z, o3c = _t3(o2, b2p, x, wlin1p, blin1p, wlin2p, blin2p, w3p)

    xl3 = jnp.pad(o3c[:, 0], (0, NPAD - N))
    xr3 = jnp.pad(o3c[:, 1], (0, NPAD - N))
    att3b = jnp.broadcast_to(att3, (16,))
    ex3, sp3 = _l3_stage1(src_p, dst_p, xl3, xr3, att3b)
    p3 = _l3_stage2(src_p, dst_p, ex3, sp3, xl3)

    pa = jnp.pad(edge_index[0], (0, ELP - E))
    pb = jnp.pad(edge_index[1], (0, ELP - E))
    na = jnp.pad(neg_edge_index[0], (0, ELP - E))
    nb = jnp.pad(neg_edge_index[1], (0, ELP - E))
    dp, dn = _loss_dots(pa, pb, na, nb, z)
    dp2 = dp[:E].reshape(1250, 128)
    dn2 = dn[:E].reshape(1250, 128)

    rl, o3 = _t4(dp2, dn2, p3, b3)
    out = o3[:N].reshape(N, 1)
    r_loss = rl[0, 0]
    return (out, r_loss, c1, c2)
